# baseline (speedup 1.0000x reference)
import functools

import jax
import jax.numpy as jnp
from jax import lax
from jax.experimental import pallas as pl
from jax.experimental.pallas import tpu as pltpu
from jax.experimental.pallas import tpu_sc as plsc

N = 10000
E = 320000
D_IN = 128
OUT = 16
H1 = 8

NW = 32
EPT = 10368
ETOT = E + N
EPAD = NW * EPT - ETOT
K1 = 64
TCH1 = EPT // K1
G1 = 54
NG1 = TCH1 // G1
K2 = 128
TCH2 = EPT // K2
NP = N + 1
NACC = 10016
RPT = NACC // 16
RW1 = 144
RW2 = 32
BLK = 1000


def _zero16():
    return jnp.zeros((16,), jnp.float32)


def _splat(v):
    return jnp.full((16,), v, jnp.int32)


def _node1_body(x_ref, w_ref, asf_ref, adf_ref, f_ref, al_ref, shift_ref, mx_ref):
    i = pl.program_id(0)
    h = jnp.dot(x_ref[...], w_ref[...], preferred_element_type=jnp.float32)
    heads = lax.broadcasted_iota(jnp.int32, (H1, H1 * OUT), 0)
    cols = lax.broadcasted_iota(jnp.int32, (H1, H1 * OUT), 1)
    blockdiag = (cols // OUT == heads).astype(jnp.float32)
    a_s = lax.dot_general(h, blockdiag * asf_ref[...],
                          (((1,), (1,)), ((), ())),
                          preferred_element_type=jnp.float32)
    a_d = lax.dot_general(h, blockdiag * adf_ref[...],
                          (((1,), (1,)), ((), ())),
                          preferred_element_type=jnp.float32)
    al = jnp.concatenate([a_s, a_d], axis=1)
    f_ref[...] = jnp.concatenate([h, al], axis=1)
    al_ref[...] = al
    cur = jnp.max(al, axis=0, keepdims=True)
    prev = jnp.where(i == 0, jnp.full_like(cur, -3.0e38), mx_ref[...])
    mx = jnp.maximum(cur, prev)
    mx_ref[...] = mx

    @pl.when(i == pl.num_programs(0) - 1)
    def _():
        g = jnp.max(mx[:, :H1] + mx[:, H1:])
        shift_ref[...] = jnp.full((1, 16), g, jnp.float32)


def _node1(x, W1, asf, adf):
    return pl.pallas_call(
        _node1_body,
        grid=(N // BLK,),
        in_specs=[
            pl.BlockSpec((BLK, D_IN), lambda i: (i, 0)),
            pl.BlockSpec((D_IN, H1 * OUT), lambda i: (0, 0)),
            pl.BlockSpec((1, H1 * OUT), lambda i: (0, 0)),
            pl.BlockSpec((1, H1 * OUT), lambda i: (0, 0)),
        ],
        out_specs=[
            pl.BlockSpec((BLK, RW1), lambda i: (i, 0)),
            pl.BlockSpec((BLK, 16), lambda i: (i, 0)),
            pl.BlockSpec((1, 16), lambda i: (0, 0)),
        ],
        out_shape=[
            jax.ShapeDtypeStruct((N, RW1), jnp.float32),
            jax.ShapeDtypeStruct((N, 16), jnp.float32),
            jax.ShapeDtypeStruct((1, 16), jnp.float32),
        ],
        scratch_shapes=[pltpu.VMEM((1, 16), jnp.float32)],
    )(x, W1, asf, adf)


def _edge1(f1p, al1p, sh1, src, dst):
    mesh = plsc.VectorSubcoreMesh(core_axis_name="c", subcore_axis_name="s")

    @functools.partial(
        pl.kernel,
        out_type=jax.ShapeDtypeStruct((2, NACC, RW1), jnp.float32),
        mesh=mesh,
        compiler_params=pltpu.CompilerParams(use_tc_tiling_on_sc=False,
                                             needs_layout_passes=False),
        scratch_types=[
            pltpu.VMEM((G1, K1), jnp.int32),
            pltpu.VMEM((G1, K1), jnp.int32),
            pltpu.VMEM((K1, RW1), jnp.float32),
            pltpu.VMEM((K1, RW1), jnp.float32),
            pltpu.VMEM((K1, 16), jnp.float32),
            pltpu.VMEM((K1, 16), jnp.float32),
            pltpu.VMEM((24, K1), jnp.float32),
            pltpu.VMEM((16,), jnp.float32),
            pltpu.VMEM_SHARED((NACC, RW1), jnp.float32),
            pltpu.SemaphoreType.DMA,
            pltpu.SemaphoreType.DMA,
            pltpu.SemaphoreType.DMA,
            pltpu.SemaphoreType.DMA,
            pltpu.SemaphoreType.DMA,
        ],
    )
    def body(f_hbm, al_hbm, sh_hbm, src_hbm, dst_hbm, out_hbm,
             idxs_v, idxd_v, hb0, hb1, ad0, ad1, wt_v, shv_v, acc,
             semh0, semh1, sema0, sema1, sems0):
        c = lax.axis_index("c")
        s = lax.axis_index("s")
        wid = s * 2 + c
        base = s * RPT

        def zrow(r, carry):
            for k9 in range(RW1 // 16):
                hb0[r, pl.ds(k9 * 16, 16)] = _zero16()
            return carry

        lax.fori_loop(0, K1, zrow, 0)
        for t in range(RPT // K1):
            pltpu.sync_copy(hb0, acc.at[pl.ds(base + t * K1, K1)])
        rem = RPT - (RPT // K1) * K1
        pltpu.sync_copy(hb0.at[pl.ds(0, rem)],
                        acc.at[pl.ds(base + (RPT // K1) * K1, rem)])
        pltpu.sync_copy(sh_hbm, shv_v)
        for r in list(range(8)) + list(range(16, 24)):
            for q in range(K1 // 16):
                wt_v[r, pl.ds(q * 16, 16)] = _zero16()
        plsc.subcore_barrier()

        shvec = shv_v[...]
        rot8 = jnp.arange(16, dtype=jnp.int32) + 8

        def issue(j, hb, semh, ad, sema):
            pltpu.async_copy(f_hbm.at[idxs_v.at[j]], hb, semh)
            pltpu.async_copy(al_hbm.at[idxd_v.at[j]], ad, sema)

        def wait_for(j, hb, semh, ad, sema):
            pltpu.make_async_copy(f_hbm.at[idxs_v.at[j]], hb, semh).wait()
            pltpu.make_async_copy(al_hbm.at[idxd_v.at[j]], ad, sema).wait()

        def compute_scatter(j, hb, ad):
            def wgrp(g4):
                lanes = jnp.arange(16, dtype=jnp.int32) + g4 * 16
                for h in range(H1):
                    asv = plsc.load_gather(hb, [lanes, _splat(H1 * OUT + h)])
                    adv = plsc.load_gather(ad, [lanes, _splat(8 + h)])
                    sv = asv + adv
                    sv = jnp.where(sv >= 0.0, sv, 0.2 * sv)
                    wt_v[8 + h, pl.ds(g4 * 16, 16)] = jnp.exp(sv - shvec)

            plsc.parallel_loop(0, K1 // 16, unroll=2)(wgrp)

            def edge(e):
                e16 = jnp.full((16,), e, jnp.int32)
                for h in range(H1):
                    wsp = plsc.load_gather(wt_v, [_splat(8 + h), e16])
                    hb[e, pl.ds(h * OUT, 16)] = hb[e, pl.ds(h * OUT, 16)] * wsp
                hb[e, pl.ds(H1 * OUT, 16)] = plsc.load_gather(wt_v, [rot8, e16])

            plsc.parallel_loop(0, K1, unroll=8)(edge)

        def group(g, gcarry):
            pltpu.sync_copy(src_hbm.at[wid, pl.ds(g * G1, G1)], idxs_v)
            pltpu.sync_copy(dst_hbm.at[wid, pl.ds(g * G1, G1)], idxd_v)
            issue(0, hb0, semh0, ad0, sema0)

            def pair(jj, carry):
                j0 = 2 * jj
                j1 = j0 + 1
                dh1 = pltpu.async_copy(f_hbm.at[idxs_v.at[j1]], hb1, semh1)
                da1 = pltpu.async_copy(al_hbm.at[idxd_v.at[j1]], ad1, sema1)
                wait_for(j0, hb0, semh0, ad0, sema0)
                compute_scatter(j0, hb0, ad0)
                ds0 = pltpu.async_copy(hb0, acc.at[idxd_v.at[j0]], sems0,
                                       add=True)
                dh1.wait()
                da1.wait()
                compute_scatter(j1, hb1, ad1)
                ds0.wait()
                jn = jnp.minimum(j0 + 2, G1 - 1)
                issue(jn, hb0, semh0, ad0, sema0)
                pltpu.sync_copy(hb1, acc.at[idxd_v.at[j1]], add=True)
                return carry

            lax.fori_loop(0, G1 // 2, pair, 0)
            wait_for(G1 - 1, hb0, semh0, ad0, sema0)
            return gcarry

        lax.fori_loop(0, NG1, group, 0)
        plsc.subcore_barrier()
        pltpu.sync_copy(acc.at[pl.ds(base, RPT)], out_hbm.at[c, pl.ds(base, RPT)])

    return body(f1p, al1p, sh1, src, dst)


def _mid_body(p_ref, w2_ref, as2_ref, ad2_ref, b1_ref,
              f2_ref, al2_ref, shift_ref, mx_ref):
    i = pl.program_id(0)
    p0 = p_ref[0]
    p1 = p_ref[1]
    num = p0[:, :H1 * OUT] + p1[:, :H1 * OUT]
    den = p0[:, H1 * OUT:H1 * OUT + H1] + p1[:, H1 * OUT:H1 * OUT + H1]
    recip = 1.0 / (den + 1e-16)
    heads = lax.broadcasted_iota(jnp.int32, (H1, H1 * OUT), 0)
    cols = lax.broadcasted_iota(jnp.int32, (H1, H1 * OUT), 1)
    blockdiag = (cols // OUT == heads).astype(jnp.float32)
    bc = jnp.dot(recip, blockdiag, preferred_element_type=jnp.float32)
    out1 = num * bc + b1_ref[...]
    out1 = jnp.where(out1 > 0.0, out1, jnp.exp(out1) - 1.0)
    h2 = jnp.dot(out1, w2_ref[...], preferred_element_type=jnp.float32)
    asv = jnp.sum(h2 * as2_ref[...], axis=1, keepdims=True)
    adv = jnp.sum(h2 * ad2_ref[...], axis=1, keepdims=True)
    al2 = jnp.concatenate(
        [asv, adv, jnp.zeros((asv.shape[0], 14), jnp.float32)], axis=1)
    f2_ref[...] = jnp.concatenate([h2, al2], axis=1)
    al2_ref[...] = al2
    cur = jnp.max(al2, axis=0, keepdims=True)
    prev = jnp.where(i == 0, jnp.full_like(cur, -3.0e38), mx_ref[...])
    mx = jnp.maximum(cur, prev)
    mx_ref[...] = mx

    @pl.when(i == pl.num_programs(0) - 1)
    def _():
        g = mx[0, 0] + mx[0, 1]
        shift_ref[...] = jnp.full((1, 16), g, jnp.float32)


def _mid(p1, W2, as2, ad2, b1r):
    return pl.pallas_call(
        _mid_body,
        grid=(N // BLK,),
        in_specs=[
            pl.BlockSpec((2, BLK, RW1), lambda i: (0, i, 0)),
            pl.BlockSpec((H1 * OUT, OUT), lambda i: (0, 0)),
            pl.BlockSpec((1, OUT), lambda i: (0, 0)),
            pl.BlockSpec((1, OUT), lambda i: (0, 0)),
            pl.BlockSpec((1, H1 * OUT), lambda i: (0, 0)),
        ],
        out_specs=[
            pl.BlockSpec((BLK, RW2), lambda i: (i, 0)),
            pl.BlockSpec((BLK, 16), lambda i: (i, 0)),
            pl.BlockSpec((1, 16), lambda i: (0, 0)),
        ],
        out_shape=[
            jax.ShapeDtypeStruct((N, RW2), jnp.float32),
            jax.ShapeDtypeStruct((N, 16), jnp.float32),
            jax.ShapeDtypeStruct((1, 16), jnp.float32),
        ],
        scratch_shapes=[pltpu.VMEM((1, 16), jnp.float32)],
    )(p1, W2, as2, ad2, b1r)


def _edge2(f2p, al2p, sh2, src, dst):
    mesh = plsc.VectorSubcoreMesh(core_axis_name="c", subcore_axis_name="s")

    @functools.partial(
        pl.kernel,
        out_type=jax.ShapeDtypeStruct((2, NACC, RW2), jnp.float32),
        mesh=mesh,
        compiler_params=pltpu.CompilerParams(use_tc_tiling_on_sc=False,
                                             needs_layout_passes=False),
        scratch_types=[
            pltpu.VMEM((TCH2, K2), jnp.int32),
            pltpu.VMEM((TCH2, K2), jnp.int32),
            pltpu.VMEM((K2, RW2), jnp.float32),
            pltpu.VMEM((K2, RW2), jnp.float32),
            pltpu.VMEM((K2, 16), jnp.float32),
            pltpu.VMEM((K2, 16), jnp.float32),
            pltpu.VMEM((24, K2), jnp.float32),
            pltpu.VMEM((16,), jnp.float32),
            pltpu.VMEM_SHARED((NACC, RW2), jnp.float32),
            pltpu.SemaphoreType.DMA,
            pltpu.SemaphoreType.DMA,
            pltpu.SemaphoreType.DMA,
            pltpu.SemaphoreType.DMA,
            pltpu.SemaphoreType.DMA,
        ],
    )
    def body(f_hbm, al_hbm, sh_hbm, src_hbm, dst_hbm, out_hbm,
             idxs_v, idxd_v, hb0, hb1, ad0, ad1, wt_v, shv_v, acc,
             semh0, semh1, sema0, sema1, sems0):
        c = lax.axis_index("c")
        s = lax.axis_index("s")
        wid = s * 2 + c
        base = s * RPT

        def zrow(r, carry):
            for k9 in range(RW2 // 16):
                hb0[r, pl.ds(k9 * 16, 16)] = _zero16()
            return carry

        lax.fori_loop(0, K2, zrow, 0)
        for t in range(RPT // K2):
            pltpu.sync_copy(hb0, acc.at[pl.ds(base + t * K2, K2)])
        rem = RPT - (RPT // K2) * K2
        pltpu.sync_copy(hb0.at[pl.ds(0, rem)],
                        acc.at[pl.ds(base + (RPT // K2) * K2, rem)])
        pltpu.sync_copy(sh_hbm, shv_v)
        for r in list(range(8)) + list(range(9, 24)):
            for q in range(K2 // 16):
                wt_v[r, pl.ds(q * 16, 16)] = _zero16()
        pltpu.sync_copy(src_hbm.at[wid], idxs_v)
        pltpu.sync_copy(dst_hbm.at[wid], idxd_v)
        plsc.subcore_barrier()

        shvec = shv_v[...]
        rot8 = jnp.arange(16, dtype=jnp.int32) + 8

        def issue(j, hb, semh, ad, sema):
            pltpu.async_copy(f_hbm.at[idxs_v.at[j]], hb, semh)
            pltpu.async_copy(al_hbm.at[idxd_v.at[j]], ad, sema)

        def wait_for(j, hb, semh, ad, sema):
            pltpu.make_async_copy(f_hbm.at[idxs_v.at[j]], hb, semh).wait()
            pltpu.make_async_copy(al_hbm.at[idxd_v.at[j]], ad, sema).wait()

        def compute_scatter(j, hb, ad):
            def wgrp(g4):
                lanes = jnp.arange(16, dtype=jnp.int32) + g4 * 16
                asv = plsc.load_gather(hb, [lanes, _splat(OUT)])
                adv = plsc.load_gather(ad, [lanes, _splat(1)])
                sv = asv + adv
                sv = jnp.where(sv >= 0.0, sv, 0.2 * sv)
                wt_v[8, pl.ds(g4 * 16, 16)] = jnp.exp(sv - shvec)

            plsc.parallel_loop(0, K2 // 16, unroll=4)(wgrp)

            def edge(e):
                e16 = jnp.full((16,), e, jnp.int32)
                wsp = plsc.load_gather(wt_v, [_splat(8), e16])
                hb[e, pl.ds(0, 16)] = hb[e, pl.ds(0, 16)] * wsp
                hb[e, pl.ds(16, 16)] = plsc.load_gather(wt_v, [rot8, e16])

            plsc.parallel_loop(0, K2, unroll=8)(edge)

        issue(0, hb0, semh0, ad0, sema0)

        def pair(jj, carry):
            j0 = 2 * jj
            j1 = j0 + 1
            dh1 = pltpu.async_copy(f_hbm.at[idxs_v.at[j1]], hb1, semh1)
            da1 = pltpu.async_copy(al_hbm.at[idxd_v.at[j1]], ad1, sema1)
            wait_for(j0, hb0, semh0, ad0, sema0)
            compute_scatter(j0, hb0, ad0)
            ds0 = pltpu.async_copy(hb0, acc.at[idxd_v.at[j0]], sems0, add=True)
            dh1.wait()
            da1.wait()
            compute_scatter(j1, hb1, ad1)
            ds0.wait()
            jn = jnp.minimum(j0 + 2, TCH2 - 1)
            issue(jn, hb0, semh0, ad0, sema0)
            pltpu.sync_copy(hb1, acc.at[idxd_v.at[j1]], add=True)
            return carry

        lax.fori_loop(0, TCH2 // 2, pair, 0)
        wait_for(TCH2 - 1, hb0, semh0, ad0, sema0)
        compute_scatter(TCH2 - 1, hb0, ad0)
        pltpu.sync_copy(hb0, acc.at[idxd_v.at[TCH2 - 1]], add=True)
        plsc.subcore_barrier()
        pltpu.sync_copy(acc.at[pl.ds(base, RPT)], out_hbm.at[c, pl.ds(base, RPT)])

    return body(f2p, al2p, sh2, src, dst)


def _final_body(p_ref, b2_ref, out_ref):
    p0 = p_ref[0]
    p1 = p_ref[1]
    num = p0[:, :OUT] + p1[:, :OUT]
    den = p0[:, OUT:OUT + 1] + p1[:, OUT:OUT + 1]
    out_ref[...] = num / (den + 1e-16) + b2_ref[...]


def _final(p2, b2r):
    return pl.pallas_call(
        _final_body,
        grid=(N // BLK,),
        in_specs=[
            pl.BlockSpec((2, BLK, RW2), lambda i: (0, i, 0)),
            pl.BlockSpec((1, OUT), lambda i: (0, 0)),
        ],
        out_specs=pl.BlockSpec((BLK, OUT), lambda i: (i, 0)),
        out_shape=jax.ShapeDtypeStruct((N, OUT), jnp.float32),
    )(p2, b2r)


@jax.jit
def kernel(x, edge_index, W1, a_src1, a_dst1, b1, W2, a_src2, a_dst2, b2):
    ei = jnp.asarray(edge_index, jnp.int32)
    loops = jnp.arange(N, dtype=jnp.int32)
    pad = jnp.full((EPAD,), N, jnp.int32)
    srcf = jnp.concatenate([ei[0], loops, pad])
    dstf = jnp.concatenate([ei[1], loops, pad])
    src1 = srcf.reshape(NW, TCH1, K1)
    dst1 = dstf.reshape(NW, TCH1, K1)
    src2 = srcf.reshape(NW, TCH2, K2)
    dst2 = dstf.reshape(NW, TCH2, K2)

    f1, al1, shift1 = _node1(x, W1,
                             a_src1.reshape(1, H1 * OUT),
                             a_dst1.reshape(1, H1 * OUT))
    padrow1 = jnp.concatenate([jnp.zeros((1, H1 * OUT), jnp.float32),
                               jnp.full((1, 16), -1e30, jnp.float32)], axis=1)
    f1p = jnp.concatenate([f1, padrow1], axis=0)
    al1p = jnp.concatenate([al1, jnp.full((1, 16), -1e30, jnp.float32)], axis=0)
    sh1 = shift1[0]
    part1 = _edge1(f1p, al1p, sh1, src1, dst1)

    f2, al2, shift2 = _mid(part1, W2,
                           a_src2.reshape(1, OUT),
                           a_dst2.reshape(1, OUT),
                           b1.reshape(1, H1 * OUT))
    padrow2 = jnp.concatenate([jnp.zeros((1, OUT), jnp.float32),
                               jnp.full((1, 16), -1e30, jnp.float32)], axis=1)
    f2p = jnp.concatenate([f2, padrow2], axis=0)
    al2p = jnp.concatenate([al2, jnp.full((1, 16), -1e30, jnp.float32)], axis=0)
    sh2 = shift2[0]
    part2 = _edge2(f2p, al2p, sh2, src2, dst2)

    return _final(part2, b2.reshape(1, OUT))


# speedup vs baseline: 1.0905x; 1.0905x over previous
import functools

import jax
import jax.numpy as jnp
from jax import lax
from jax.experimental import pallas as pl
from jax.experimental.pallas import tpu as pltpu
from jax.experimental.pallas import tpu_sc as plsc

N = 10000
E = 320000
D_IN = 128
OUT = 16
H1 = 8

NW = 32
EPT = 10368
ETOT = E + N
EPAD = NW * EPT - ETOT
K1 = 64
TCH1 = EPT // K1
G1 = 54
NG1 = TCH1 // G1
K2 = 128
TCH2 = EPT // K2
NP = N + 1
NACC = 10016
RPT = NACC // 16
RW1 = 144
RW2 = 32
BLK = 1000


def _zero16():
    return jnp.zeros((16,), jnp.float32)


def _splat(v):
    return jnp.full((16,), v, jnp.int32)


def _node1_body(x_ref, w_ref, asf_ref, adf_ref, f_ref, al_ref, shift_ref, mx_ref):
    i = pl.program_id(0)
    h = jnp.dot(x_ref[...], w_ref[...], preferred_element_type=jnp.float32)
    heads = lax.broadcasted_iota(jnp.int32, (H1, H1 * OUT), 0)
    cols = lax.broadcasted_iota(jnp.int32, (H1, H1 * OUT), 1)
    blockdiag = (cols // OUT == heads).astype(jnp.float32)
    a_s = lax.dot_general(h, blockdiag * asf_ref[...],
                          (((1,), (1,)), ((), ())),
                          preferred_element_type=jnp.float32)
    a_d = lax.dot_general(h, blockdiag * adf_ref[...],
                          (((1,), (1,)), ((), ())),
                          preferred_element_type=jnp.float32)
    al = jnp.concatenate([a_s, a_d], axis=1)
    f_ref[...] = jnp.concatenate([h, al], axis=1)
    al_ref[...] = al
    cur = jnp.max(al, axis=0, keepdims=True)
    prev = jnp.where(i == 0, jnp.full_like(cur, -3.0e38), mx_ref[...])
    mx = jnp.maximum(cur, prev)
    mx_ref[...] = mx

    @pl.when(i == pl.num_programs(0) - 1)
    def _():
        g = jnp.max(mx[:, :H1] + mx[:, H1:])
        shift_ref[...] = jnp.full((1, 16), g, jnp.float32)


def _node1(x, W1, asf, adf):
    return pl.pallas_call(
        _node1_body,
        grid=(N // BLK,),
        in_specs=[
            pl.BlockSpec((BLK, D_IN), lambda i: (i, 0)),
            pl.BlockSpec((D_IN, H1 * OUT), lambda i: (0, 0)),
            pl.BlockSpec((1, H1 * OUT), lambda i: (0, 0)),
            pl.BlockSpec((1, H1 * OUT), lambda i: (0, 0)),
        ],
        out_specs=[
            pl.BlockSpec((BLK, RW1), lambda i: (i, 0)),
            pl.BlockSpec((BLK, 16), lambda i: (i, 0)),
            pl.BlockSpec((1, 16), lambda i: (0, 0)),
        ],
        out_shape=[
            jax.ShapeDtypeStruct((N, RW1), jnp.float32),
            jax.ShapeDtypeStruct((N, 16), jnp.float32),
            jax.ShapeDtypeStruct((1, 16), jnp.float32),
        ],
        scratch_shapes=[pltpu.VMEM((1, 16), jnp.float32)],
    )(x, W1, asf, adf)


def _edge1(f1p, al1p, sh1, src, dst):
    mesh = plsc.VectorSubcoreMesh(core_axis_name="c", subcore_axis_name="s")

    @functools.partial(
        pl.kernel,
        out_type=jax.ShapeDtypeStruct((2, NACC, RW1), jnp.float32),
        mesh=mesh,
        compiler_params=pltpu.CompilerParams(use_tc_tiling_on_sc=False,
                                             needs_layout_passes=False),
        scratch_types=[
            pltpu.VMEM((G1, K1), jnp.int32),
            pltpu.VMEM((G1, K1), jnp.int32),
            pltpu.VMEM((K1, RW1), jnp.float32),
            pltpu.VMEM((K1, RW1), jnp.float32),
            pltpu.VMEM((K1, 16), jnp.float32),
            pltpu.VMEM((K1, 16), jnp.float32),
            pltpu.VMEM((24, K1), jnp.float32),
            pltpu.VMEM((16,), jnp.float32),
            pltpu.VMEM_SHARED((NACC, RW1), jnp.float32),
            pltpu.SemaphoreType.DMA,
            pltpu.SemaphoreType.DMA,
            pltpu.SemaphoreType.DMA,
            pltpu.SemaphoreType.DMA,
        ],
    )
    def body(f_hbm, al_hbm, sh_hbm, src_hbm, dst_hbm, out_hbm,
             idxs_v, idxd_v, hb0, hb1, ad0, ad1, wt_v, shv_v, acc,
             semh0, semh1, sema0, sema1):
        c = lax.axis_index("c")
        s = lax.axis_index("s")
        wid = s * 2 + c
        base = s * RPT

        def zrow(r, carry):
            for k9 in range(RW1 // 16):
                hb0[r, pl.ds(k9 * 16, 16)] = _zero16()
            return carry

        lax.fori_loop(0, K1, zrow, 0)
        for t in range(RPT // K1):
            pltpu.sync_copy(hb0, acc.at[pl.ds(base + t * K1, K1)])
        rem = RPT - (RPT // K1) * K1
        pltpu.sync_copy(hb0.at[pl.ds(0, rem)],
                        acc.at[pl.ds(base + (RPT // K1) * K1, rem)])
        pltpu.sync_copy(sh_hbm, shv_v)
        for r in list(range(8)) + list(range(16, 24)):
            for q in range(K1 // 16):
                wt_v[r, pl.ds(q * 16, 16)] = _zero16()
        plsc.subcore_barrier()

        shvec = shv_v[...]
        rot8 = jnp.arange(16, dtype=jnp.int32) + 8

        def issue(j, hb, semh, ad, sema):
            pltpu.async_copy(f_hbm.at[idxs_v.at[j]], hb, semh)
            pltpu.async_copy(al_hbm.at[idxd_v.at[j]], ad, sema)

        def wait_for(j, hb, semh, ad, sema):
            pltpu.make_async_copy(f_hbm.at[idxs_v.at[j]], hb, semh).wait()
            pltpu.make_async_copy(al_hbm.at[idxd_v.at[j]], ad, sema).wait()

        def compute_scatter(j, hb, ad):
            def wgrp(g4):
                lanes = jnp.arange(16, dtype=jnp.int32) + g4 * 16
                for h in range(H1):
                    asv = plsc.load_gather(hb, [lanes, _splat(H1 * OUT + h)])
                    adv = plsc.load_gather(ad, [lanes, _splat(8 + h)])
                    sv = asv + adv
                    sv = jnp.where(sv >= 0.0, sv, 0.2 * sv)
                    wt_v[8 + h, pl.ds(g4 * 16, 16)] = jnp.exp(sv - shvec)

            plsc.parallel_loop(0, K1 // 16, unroll=2)(wgrp)

            def edge(e):
                e16 = jnp.full((16,), e, jnp.int32)
                for h in range(H1):
                    wsp = plsc.load_gather(wt_v, [_splat(8 + h), e16])
                    hb[e, pl.ds(h * OUT, 16)] = hb[e, pl.ds(h * OUT, 16)] * wsp
                hb[e, pl.ds(H1 * OUT, 16)] = plsc.load_gather(wt_v, [rot8, e16])

            plsc.parallel_loop(0, K1, unroll=8)(edge)
            pltpu.sync_copy(hb, acc.at[idxd_v.at[j]], add=True)

        def group(g, gcarry):
            pltpu.sync_copy(src_hbm.at[wid, pl.ds(g * G1, G1)], idxs_v)
            pltpu.sync_copy(dst_hbm.at[wid, pl.ds(g * G1, G1)], idxd_v)
            issue(0, hb0, semh0, ad0, sema0)

            def pair(jj, carry):
                j0 = 2 * jj
                j1 = j0 + 1
                dh1 = pltpu.async_copy(f_hbm.at[idxs_v.at[j1]], hb1, semh1)
                da1 = pltpu.async_copy(al_hbm.at[idxd_v.at[j1]], ad1, sema1)
                wait_for(j0, hb0, semh0, ad0, sema0)
                compute_scatter(j0, hb0, ad0)
                jn = jnp.minimum(j0 + 2, G1 - 1)
                issue(jn, hb0, semh0, ad0, sema0)
                dh1.wait()
                da1.wait()
                compute_scatter(j1, hb1, ad1)
                return carry

            lax.fori_loop(0, G1 // 2, pair, 0)
            wait_for(G1 - 1, hb0, semh0, ad0, sema0)
            return gcarry

        lax.fori_loop(0, NG1, group, 0)
        plsc.subcore_barrier()
        pltpu.sync_copy(acc.at[pl.ds(base, RPT)], out_hbm.at[c, pl.ds(base, RPT)])

    return body(f1p, al1p, sh1, src, dst)


def _mid_body(p_ref, w2_ref, as2_ref, ad2_ref, b1_ref,
              f2_ref, al2_ref, shift_ref, mx_ref):
    i = pl.program_id(0)
    p0 = p_ref[0]
    p1 = p_ref[1]
    num = p0[:, :H1 * OUT] + p1[:, :H1 * OUT]
    den = p0[:, H1 * OUT:H1 * OUT + H1] + p1[:, H1 * OUT:H1 * OUT + H1]
    recip = 1.0 / (den + 1e-16)
    heads = lax.broadcasted_iota(jnp.int32, (H1, H1 * OUT), 0)
    cols = lax.broadcasted_iota(jnp.int32, (H1, H1 * OUT), 1)
    blockdiag = (cols // OUT == heads).astype(jnp.float32)
    bc = jnp.dot(recip, blockdiag, preferred_element_type=jnp.float32)
    out1 = num * bc + b1_ref[...]
    out1 = jnp.where(out1 > 0.0, out1, jnp.exp(out1) - 1.0)
    h2 = jnp.dot(out1, w2_ref[...], preferred_element_type=jnp.float32)
    asv = jnp.sum(h2 * as2_ref[...], axis=1, keepdims=True)
    adv = jnp.sum(h2 * ad2_ref[...], axis=1, keepdims=True)
    al2 = jnp.concatenate(
        [asv, adv, jnp.zeros((asv.shape[0], 14), jnp.float32)], axis=1)
    f2_ref[...] = jnp.concatenate([h2, al2], axis=1)
    al2_ref[...] = al2
    cur = jnp.max(al2, axis=0, keepdims=True)
    prev = jnp.where(i == 0, jnp.full_like(cur, -3.0e38), mx_ref[...])
    mx = jnp.maximum(cur, prev)
    mx_ref[...] = mx

    @pl.when(i == pl.num_programs(0) - 1)
    def _():
        g = mx[0, 0] + mx[0, 1]
        shift_ref[...] = jnp.full((1, 16), g, jnp.float32)


def _mid(p1, W2, as2, ad2, b1r):
    return pl.pallas_call(
        _mid_body,
        grid=(N // BLK,),
        in_specs=[
            pl.BlockSpec((2, BLK, RW1), lambda i: (0, i, 0)),
            pl.BlockSpec((H1 * OUT, OUT), lambda i: (0, 0)),
            pl.BlockSpec((1, OUT), lambda i: (0, 0)),
            pl.BlockSpec((1, OUT), lambda i: (0, 0)),
            pl.BlockSpec((1, H1 * OUT), lambda i: (0, 0)),
        ],
        out_specs=[
            pl.BlockSpec((BLK, RW2), lambda i: (i, 0)),
            pl.BlockSpec((BLK, 16), lambda i: (i, 0)),
            pl.BlockSpec((1, 16), lambda i: (0, 0)),
        ],
        out_shape=[
            jax.ShapeDtypeStruct((N, RW2), jnp.float32),
            jax.ShapeDtypeStruct((N, 16), jnp.float32),
            jax.ShapeDtypeStruct((1, 16), jnp.float32),
        ],
        scratch_shapes=[pltpu.VMEM((1, 16), jnp.float32)],
    )(p1, W2, as2, ad2, b1r)


def _edge2(f2p, al2p, sh2, src, dst):
    mesh = plsc.VectorSubcoreMesh(core_axis_name="c", subcore_axis_name="s")

    @functools.partial(
        pl.kernel,
        out_type=jax.ShapeDtypeStruct((2, NACC, RW2), jnp.float32),
        mesh=mesh,
        compiler_params=pltpu.CompilerParams(use_tc_tiling_on_sc=False,
                                             needs_layout_passes=False),
        scratch_types=[
            pltpu.VMEM((TCH2, K2), jnp.int32),
            pltpu.VMEM((TCH2, K2), jnp.int32),
            pltpu.VMEM((K2, RW2), jnp.float32),
            pltpu.VMEM((K2, RW2), jnp.float32),
            pltpu.VMEM((K2, 16), jnp.float32),
            pltpu.VMEM((K2, 16), jnp.float32),
            pltpu.VMEM((24, K2), jnp.float32),
            pltpu.VMEM((16,), jnp.float32),
            pltpu.VMEM_SHARED((NACC, RW2), jnp.float32),
            pltpu.SemaphoreType.DMA,
            pltpu.SemaphoreType.DMA,
            pltpu.SemaphoreType.DMA,
            pltpu.SemaphoreType.DMA,
        ],
    )
    def body(f_hbm, al_hbm, sh_hbm, src_hbm, dst_hbm, out_hbm,
             idxs_v, idxd_v, hb0, hb1, ad0, ad1, wt_v, shv_v, acc,
             semh0, semh1, sema0, sema1):
        c = lax.axis_index("c")
        s = lax.axis_index("s")
        wid = s * 2 + c
        base = s * RPT

        def zrow(r, carry):
            for k9 in range(RW2 // 16):
                hb0[r, pl.ds(k9 * 16, 16)] = _zero16()
            return carry

        lax.fori_loop(0, K2, zrow, 0)
        for t in range(RPT // K2):
            pltpu.sync_copy(hb0, acc.at[pl.ds(base + t * K2, K2)])
        rem = RPT - (RPT // K2) * K2
        pltpu.sync_copy(hb0.at[pl.ds(0, rem)],
                        acc.at[pl.ds(base + (RPT // K2) * K2, rem)])
        pltpu.sync_copy(sh_hbm, shv_v)
        for r in list(range(8)) + list(range(9, 24)):
            for q in range(K2 // 16):
                wt_v[r, pl.ds(q * 16, 16)] = _zero16()
        pltpu.sync_copy(src_hbm.at[wid], idxs_v)
        pltpu.sync_copy(dst_hbm.at[wid], idxd_v)
        plsc.subcore_barrier()

        shvec = shv_v[...]
        rot8 = jnp.arange(16, dtype=jnp.int32) + 8

        def issue(j, hb, semh, ad, sema):
            pltpu.async_copy(f_hbm.at[idxs_v.at[j]], hb, semh)
            pltpu.async_copy(al_hbm.at[idxd_v.at[j]], ad, sema)

        def wait_for(j, hb, semh, ad, sema):
            pltpu.make_async_copy(f_hbm.at[idxs_v.at[j]], hb, semh).wait()
            pltpu.make_async_copy(al_hbm.at[idxd_v.at[j]], ad, sema).wait()

        def compute_scatter(j, hb, ad):
            def wgrp(g4):
                lanes = jnp.arange(16, dtype=jnp.int32) + g4 * 16
                asv = plsc.load_gather(hb, [lanes, _splat(OUT)])
                adv = plsc.load_gather(ad, [lanes, _splat(1)])
                sv = asv + adv
                sv = jnp.where(sv >= 0.0, sv, 0.2 * sv)
                wt_v[8, pl.ds(g4 * 16, 16)] = jnp.exp(sv - shvec)

            plsc.parallel_loop(0, K2 // 16, unroll=4)(wgrp)

            def edge(e):
                e16 = jnp.full((16,), e, jnp.int32)
                wsp = plsc.load_gather(wt_v, [_splat(8), e16])
                hb[e, pl.ds(0, 16)] = hb[e, pl.ds(0, 16)] * wsp
                hb[e, pl.ds(16, 16)] = plsc.load_gather(wt_v, [rot8, e16])

            plsc.parallel_loop(0, K2, unroll=8)(edge)
            pltpu.sync_copy(hb, acc.at[idxd_v.at[j]], add=True)

        issue(0, hb0, semh0, ad0, sema0)

        def pair(jj, carry):
            j0 = 2 * jj
            j1 = j0 + 1
            dh1 = pltpu.async_copy(f_hbm.at[idxs_v.at[j1]], hb1, semh1)
            da1 = pltpu.async_copy(al_hbm.at[idxd_v.at[j1]], ad1, sema1)
            wait_for(j0, hb0, semh0, ad0, sema0)
            compute_scatter(j0, hb0, ad0)
            jn = jnp.minimum(j0 + 2, TCH2 - 1)
            issue(jn, hb0, semh0, ad0, sema0)
            dh1.wait()
            da1.wait()
            compute_scatter(j1, hb1, ad1)
            return carry

        lax.fori_loop(0, TCH2 // 2, pair, 0)
        wait_for(TCH2 - 1, hb0, semh0, ad0, sema0)
        compute_scatter(TCH2 - 1, hb0, ad0)
        plsc.subcore_barrier()
        pltpu.sync_copy(acc.at[pl.ds(base, RPT)], out_hbm.at[c, pl.ds(base, RPT)])

    return body(f2p, al2p, sh2, src, dst)


def _final_body(p_ref, b2_ref, out_ref):
    p0 = p_ref[0]
    p1 = p_ref[1]
    num = p0[:, :OUT] + p1[:, :OUT]
    den = p0[:, OUT:OUT + 1] + p1[:, OUT:OUT + 1]
    out_ref[...] = num / (den + 1e-16) + b2_ref[...]


def _final(p2, b2r):
    return pl.pallas_call(
        _final_body,
        grid=(N // BLK,),
        in_specs=[
            pl.BlockSpec((2, BLK, RW2), lambda i: (0, i, 0)),
            pl.BlockSpec((1, OUT), lambda i: (0, 0)),
        ],
        out_specs=pl.BlockSpec((BLK, OUT), lambda i: (i, 0)),
        out_shape=jax.ShapeDtypeStruct((N, OUT), jnp.float32),
    )(p2, b2r)


@jax.jit
def kernel(x, edge_index, W1, a_src1, a_dst1, b1, W2, a_src2, a_dst2, b2):
    ei = jnp.asarray(edge_index, jnp.int32)
    loops = jnp.arange(N, dtype=jnp.int32)
    pad = jnp.full((EPAD,), N, jnp.int32)
    srcf = jnp.concatenate([ei[0], loops, pad])
    dstf = jnp.concatenate([ei[1], loops, pad])
    src1 = srcf.reshape(NW, TCH1, K1)
    dst1 = dstf.reshape(NW, TCH1, K1)
    src2 = srcf.reshape(NW, TCH2, K2)
    dst2 = dstf.reshape(NW, TCH2, K2)

    f1, al1, shift1 = _node1(x, W1,
                             a_src1.reshape(1, H1 * OUT),
                             a_dst1.reshape(1, H1 * OUT))
    padrow1 = jnp.concatenate([jnp.zeros((1, H1 * OUT), jnp.float32),
                               jnp.full((1, 16), -1e30, jnp.float32)], axis=1)
    f1p = jnp.concatenate([f1, padrow1], axis=0)
    al1p = jnp.concatenate([al1, jnp.full((1, 16), -1e30, jnp.float32)], axis=0)
    sh1 = shift1[0]
    part1 = _edge1(f1p, al1p, sh1, src1, dst1)

    f2, al2, shift2 = _mid(part1, W2,
                           a_src2.reshape(1, OUT),
                           a_dst2.reshape(1, OUT),
                           b1.reshape(1, H1 * OUT))
    padrow2 = jnp.concatenate([jnp.zeros((1, OUT), jnp.float32),
                               jnp.full((1, 16), -1e30, jnp.float32)], axis=1)
    f2p = jnp.concatenate([f2, padrow2], axis=0)
    al2p = jnp.concatenate([al2, jnp.full((1, 16), -1e30, jnp.float32)], axis=0)
    sh2 = shift2[0]
    part2 = _edge2(f2p, al2p, sh2, src2, dst2)

    return _final(part2, b2.reshape(1, OUT))
